# baseline (speedup 1.0000x reference)
import jax
import jax.numpy as jnp
from jax.experimental import pallas as pl
from jax.experimental.pallas import tpu as pltpu


def _mm_kernel(x_ref, t_ref, o_ref):
    o_ref[...] = jnp.dot(x_ref[...], t_ref[...],
                         preferred_element_type=jnp.float32)


def kernel(inputs, table):
    B, V = inputs.shape
    _, D = table.shape
    return pl.pallas_call(
        _mm_kernel,
        in_specs=[
            pl.BlockSpec(memory_space=pltpu.MemorySpace.VMEM),
            pl.BlockSpec(memory_space=pltpu.MemorySpace.VMEM),
        ],
        out_specs=pl.BlockSpec(memory_space=pltpu.MemorySpace.VMEM),
        out_shape=jax.ShapeDtypeStruct((B, D), jnp.float32),
    )(inputs, table)
